# baseline (speedup 1.0000x reference)
import functools

import jax
import jax.numpy as jnp
from jax.experimental import pallas as pl
from jax.experimental.pallas import tpu as pltpu
from jax.experimental.pallas import tpu_sc as plsc

_EPS = 1e-5
_NW = 32
_IDXW = 128
_CH = 2
_CE = _CH * _IDXW
_LANES = 128


def _bn_body(x_ref, g_ref, b_ref, z_ref):
    x = x_ref[...]
    n = x.shape[0]
    mu = jnp.sum(x, axis=0, keepdims=True) / n
    xc = x - mu
    var = jnp.sum(xc * xc, axis=0, keepdims=True) / n
    z_ref[...] = g_ref[...] * xc * jax.lax.rsqrt(var + _EPS) + b_ref[...]


def _batchnorm(x, gamma, beta):
    n, d = x.shape
    return pl.pallas_call(
        _bn_body,
        out_shape=jax.ShapeDtypeStruct((n, d), jnp.float32),
    )(x, gamma.reshape(1, d), beta.reshape(1, d))


def _mlp_body(xi_ref, xj_ref, w1_ref, b1_ref, w2_ref, b2_ref, w3_ref, b3_ref,
              o_ref, *, last_relu):
    din = w1_ref.shape[0] // 2
    dout = w3_ref.shape[1]
    be = xi_ref.shape[0]

    def mm(a, w):
        return jnp.dot(a.astype(jnp.bfloat16), w,
                       preferred_element_type=jnp.float32)

    h = jnp.concatenate([xi_ref[...][:, :din], xj_ref[...][:, :din]], axis=1)
    h = jnp.maximum(mm(h, w1_ref[...]) + b1_ref[...], 0.0)
    h = jnp.maximum(mm(h, w2_ref[...]) + b2_ref[...], 0.0)
    o = mm(h, w3_ref[...]) + b3_ref[...]
    if last_relu:
        o = jnp.maximum(o, 0.0)
    if dout < _LANES:
        o = jnp.concatenate(
            [o, jnp.full((be, 8), 1.0, jnp.float32),
             jnp.zeros((be, _LANES - dout - 8), jnp.float32)], axis=1)
    o_ref[...] = o


def _edge_mlp(xi, xj, w1, b1, w2, b2, w3, b3, last_relu, be=512):
    e, din = xi.shape
    big = w1.shape[1]
    dout = w3.shape[1]
    w1 = w1.astype(jnp.bfloat16)
    w2 = w2.astype(jnp.bfloat16)
    w3 = w3.astype(jnp.bfloat16)
    b1 = b1.reshape(1, big)
    b2 = b2.reshape(1, big)
    b3 = b3.reshape(1, dout)
    grid = e // be
    wspec = lambda a: pl.BlockSpec(a.shape, lambda i: (0, 0))
    return pl.pallas_call(
        functools.partial(_mlp_body, last_relu=last_relu),
        grid=(grid,),
        in_specs=[
            pl.BlockSpec((be, din), lambda i: (i, 0)),
            pl.BlockSpec((be, din), lambda i: (i, 0)),
            wspec(w1), wspec(b1), wspec(w2), wspec(b2), wspec(w3), wspec(b3),
        ],
        out_specs=pl.BlockSpec((be, _LANES), lambda i: (i, 0)),
        out_shape=jax.ShapeDtypeStruct((e, _LANES), jnp.float32),
        compiler_params=pltpu.CompilerParams(
            dimension_semantics=("arbitrary",)),
    )(xi, xj, w1, b1, w2, b2, w3, b3)


def _combine_body(p_ref, c_ref, y_ref, *, keep):
    p = p_ref[...]
    psum = p[0] + p[1]
    c = c_ref[...]
    cnt = (c[0] + c[1])[:, 64:65]
    y = psum / jnp.maximum(cnt, 1.0)
    if keep < _LANES:
        y = jnp.concatenate(
            [y[:, :keep], jnp.zeros((y.shape[0], _LANES - keep), y.dtype)],
            axis=1)
    y_ref[...] = y


def _combine(parts, cnt_parts, keep):
    _, n, d = parts.shape
    return pl.pallas_call(
        functools.partial(_combine_body, keep=keep),
        out_shape=jax.ShapeDtypeStruct((n, d), jnp.float32),
    )(parts, cnt_parts)


def _sc_gather_pair(table, dst3d, src3d):
    n, d = table.shape
    n_chunks = dst3d.shape[0]
    e = n_chunks * _CE
    k_iters = -(-n_chunks // _NW)
    mesh = plsc.VectorSubcoreMesh(core_axis_name="c", subcore_axis_name="s")

    def body(table_ref, dst_ref, src_ref, xi_ref, xj_ref,
             idx_i, idx_j, rows_i, rows_j, sem_i, sem_j):
        cc = jax.lax.axis_index("c")
        ss = jax.lax.axis_index("s")
        wid = ss * 2 + cc

        def step(k, carry):
            g = wid + _NW * k

            @pl.when(g < n_chunks)
            def _():
                pltpu.sync_copy(dst_ref.at[g], idx_i)
                pltpu.sync_copy(src_ref.at[g], idx_j)
                cps = []
                for j in range(_CH):
                    cps.append(pltpu.async_copy(
                        table_ref.at[idx_i.at[j]],
                        rows_i.at[pl.ds(j * _IDXW, _IDXW)], sem_i))
                    cps.append(pltpu.async_copy(
                        table_ref.at[idx_j.at[j]],
                        rows_j.at[pl.ds(j * _IDXW, _IDXW)], sem_j))
                for cp in cps:
                    cp.wait()
                pltpu.sync_copy(rows_i, xi_ref.at[pl.ds(g * _CE, _CE)])
                pltpu.sync_copy(rows_j, xj_ref.at[pl.ds(g * _CE, _CE)])
            return carry

        jax.lax.fori_loop(0, k_iters, step, 0)

    f = pl.kernel(
        body,
        out_type=[jax.ShapeDtypeStruct((e, d), jnp.float32),
                  jax.ShapeDtypeStruct((e, d), jnp.float32)],
        mesh=mesh,
        scratch_types=[
            pltpu.VMEM((_CH, _IDXW), jnp.int32),
            pltpu.VMEM((_CH, _IDXW), jnp.int32),
            pltpu.VMEM((_CE, d), jnp.float32),
            pltpu.VMEM((_CE, d), jnp.float32),
            pltpu.SemaphoreType.DMA,
            pltpu.SemaphoreType.DMA,
        ],
    )
    return f(table, dst3d, src3d)


def _sc_scatter(msg, dst3d, n):
    e, d = msg.shape
    n_chunks = e // _CE
    k_iters = -(-n_chunks // _NW)
    rpt = (n // 16) // 8 * 8
    tail = n - 16 * rpt
    zch = 208
    assert rpt % zch == 0 and tail <= zch
    mesh = plsc.VectorSubcoreMesh(core_axis_name="c", subcore_axis_name="s")

    def body(msg_ref, dst_ref, z_ref, part_ref, idx_v, rows_v, accum):
        cc = jax.lax.axis_index("c")
        ss = jax.lax.axis_index("s")
        wid = ss * 2 + cc

        def striped(fn):
            for i in range(rpt // zch):
                fn(ss * rpt + i * zch, zch)
            if tail:
                @pl.when(ss == 15)
                def _():
                    fn(16 * rpt, tail)

        pltpu.sync_copy(z_ref, rows_v.at[pl.ds(0, zch)])
        striped(lambda at, ln: pltpu.sync_copy(
            rows_v.at[pl.ds(0, ln)], accum.at[pl.ds(at, ln)]))
        plsc.subcore_barrier()

        def step(k, carry):
            g = wid + _NW * k

            @pl.when(g < n_chunks)
            def _():
                pltpu.sync_copy(dst_ref.at[g], idx_v)
                pltpu.sync_copy(msg_ref.at[pl.ds(g * _CE, _CE)], rows_v)
                for j in range(_CH):
                    pltpu.sync_copy(rows_v.at[pl.ds(j * _IDXW, _IDXW)],
                                    accum.at[idx_v.at[j]], add=True)
            return carry

        jax.lax.fori_loop(0, k_iters, step, 0)
        plsc.subcore_barrier()

        def wb(at, ln):
            pltpu.sync_copy(accum.at[pl.ds(at, ln)], rows_v.at[pl.ds(0, ln)])
            pltpu.sync_copy(rows_v.at[pl.ds(0, ln)],
                            part_ref.at[pl.ds(cc * n + at, ln)])
        striped(wb)

    f = pl.kernel(
        body,
        out_type=[jax.ShapeDtypeStruct((2 * n, d), jnp.float32)],
        mesh=mesh,
        scratch_types=[
            pltpu.VMEM((_CH, _IDXW), jnp.int32),
            pltpu.VMEM((_CE, d), jnp.float32),
            pltpu.VMEM_SHARED((n, d), jnp.float32),
        ],
    )
    (out,) = f(msg, dst3d, jnp.zeros((zch, d), jnp.float32))
    return out.reshape(2, n, d)


def kernel(x, edge_index, bn_gamma, bn_beta, eW1, eb1, eW2, eb2, eW3, eb3,
           dW1, db1, dW2, db2, dW3, db3):
    n, d = x.shape
    hid = eW3.shape[1]
    src = edge_index[0].reshape(-1, _CH, _IDXW)
    dst = edge_index[1].reshape(-1, _CH, _IDXW)

    eW1p = jnp.concatenate([eW1[:d] - eW1[d:], eW1[d:]], axis=0)
    dW1p = jnp.concatenate([dW1[:hid] - dW1[hid:], dW1[hid:]], axis=0)

    z = _batchnorm(x, bn_gamma, bn_beta)

    xi, xj = _sc_gather_pair(z, dst, src)
    m1 = _edge_mlp(xi, xj, eW1p, eb1, eW2, eb2, eW3, eb3, last_relu=True)
    parts1 = _sc_scatter(m1, dst, n)
    y = _combine(parts1, parts1, keep=hid)

    yi, yj = _sc_gather_pair(y, dst, src)
    m2 = _edge_mlp(yi, yj, dW1p, db1, dW2, db2, dW3, db3, last_relu=False)
    parts2 = _sc_scatter(m2, dst, n)
    return _combine(parts2, parts1, keep=d)


# speedup vs baseline: 1.4186x; 1.4186x over previous
import functools

import jax
import jax.numpy as jnp
from jax.experimental import pallas as pl
from jax.experimental.pallas import tpu as pltpu
from jax.experimental.pallas import tpu_sc as plsc

_EPS = 1e-5
_NW = 32
_IDXW = 128
_CH = 2
_CE = _CH * _IDXW
_LANES = 128


def _bn_body(x_ref, g_ref, b_ref, z_ref):
    x = x_ref[...]
    n = x.shape[0]
    mu = jnp.sum(x, axis=0, keepdims=True) / n
    xc = x - mu
    var = jnp.sum(xc * xc, axis=0, keepdims=True) / n
    z_ref[...] = g_ref[...] * xc * jax.lax.rsqrt(var + _EPS) + b_ref[...]


def _batchnorm(x, gamma, beta):
    n, d = x.shape
    return pl.pallas_call(
        _bn_body,
        out_shape=jax.ShapeDtypeStruct((n, d), jnp.float32),
    )(x, gamma.reshape(1, d), beta.reshape(1, d))


def _mlp_body(xi_ref, xj_ref, w1_ref, b1_ref, w2_ref, b2_ref, w3_ref, b3_ref,
              o_ref, *, last_relu):
    din = w1_ref.shape[0] // 2
    dout = w3_ref.shape[1]
    be = xi_ref.shape[0]

    def mm(a, w):
        if a.dtype != jnp.bfloat16:
            a = a.astype(jnp.bfloat16)
        return jnp.dot(a, w, preferred_element_type=jnp.float32)

    h = jnp.concatenate([xi_ref[...][:, :din], xj_ref[...][:, :din]], axis=1)
    h = jnp.maximum(mm(h, w1_ref[...]) + b1_ref[...], 0.0)
    h = jnp.maximum(mm(h, w2_ref[...]) + b2_ref[...], 0.0)
    o = mm(h, w3_ref[...]) + b3_ref[...]
    if last_relu:
        o = jnp.maximum(o, 0.0)
    if dout < _LANES:
        o = jnp.concatenate(
            [o, jnp.full((be, 8), 1.0, jnp.float32),
             jnp.zeros((be, _LANES - dout - 8), jnp.float32)], axis=1)
    o_ref[...] = o


def _edge_mlp(xi, xj, w1, b1, w2, b2, w3, b3, last_relu, be=2048):
    e, din = xi.shape
    big = w1.shape[1]
    dout = w3.shape[1]
    w1 = w1.astype(jnp.bfloat16)
    w2 = w2.astype(jnp.bfloat16)
    w3 = w3.astype(jnp.bfloat16)
    b1 = b1.reshape(1, big)
    b2 = b2.reshape(1, big)
    b3 = b3.reshape(1, dout)
    grid = e // be
    wspec = lambda a: pl.BlockSpec(a.shape, lambda i: (0, 0))
    return pl.pallas_call(
        functools.partial(_mlp_body, last_relu=last_relu),
        grid=(grid,),
        in_specs=[
            pl.BlockSpec((be, din), lambda i: (i, 0)),
            pl.BlockSpec((be, din), lambda i: (i, 0)),
            wspec(w1), wspec(b1), wspec(w2), wspec(b2), wspec(w3), wspec(b3),
        ],
        out_specs=pl.BlockSpec((be, _LANES), lambda i: (i, 0)),
        out_shape=jax.ShapeDtypeStruct((e, _LANES), jnp.float32),
        compiler_params=pltpu.CompilerParams(
            dimension_semantics=("arbitrary",)),
    )(xi, xj, w1, b1, w2, b2, w3, b3)


def _combine_body(p_ref, c_ref, y_ref, *, keep):
    p = p_ref[...]
    psum = p[0] + p[1]
    c = c_ref[...]
    cnt = (c[0] + c[1])[:, 64:65]
    y = psum / jnp.maximum(cnt, 1.0)
    if keep < _LANES:
        y = jnp.concatenate(
            [y[:, :keep], jnp.zeros((y.shape[0], _LANES - keep), y.dtype)],
            axis=1)
    y_ref[...] = y.astype(y_ref.dtype)


def _combine(parts, cnt_parts, keep, out_dtype=jnp.float32):
    _, n, d = parts.shape
    return pl.pallas_call(
        functools.partial(_combine_body, keep=keep),
        out_shape=jax.ShapeDtypeStruct((n, d), out_dtype),
    )(parts, cnt_parts)


def _sc_gather_pair(table, dst3d, src3d):
    n, d = table.shape
    n_chunks = dst3d.shape[0]
    e = n_chunks * _CE
    k_iters = -(-n_chunks // _NW)
    mesh = plsc.VectorSubcoreMesh(core_axis_name="c", subcore_axis_name="s")

    def body(table_ref, dst_ref, src_ref, xi_ref, xj_ref,
             idx_i, idx_j, rows_i, rows_j, sem_i, sem_j):
        cc = jax.lax.axis_index("c")
        ss = jax.lax.axis_index("s")
        wid = ss * 2 + cc

        def step(k, carry):
            g = wid + _NW * k

            @pl.when(g < n_chunks)
            def _():
                pltpu.sync_copy(dst_ref.at[g], idx_i)
                pltpu.sync_copy(src_ref.at[g], idx_j)
                cps = []
                for j in range(_CH):
                    cps.append(pltpu.async_copy(
                        table_ref.at[idx_i.at[j]],
                        rows_i.at[pl.ds(j * _IDXW, _IDXW)], sem_i))
                    cps.append(pltpu.async_copy(
                        table_ref.at[idx_j.at[j]],
                        rows_j.at[pl.ds(j * _IDXW, _IDXW)], sem_j))
                for cp in cps:
                    cp.wait()
                pltpu.sync_copy(rows_i, xi_ref.at[pl.ds(g * _CE, _CE)])
                pltpu.sync_copy(rows_j, xj_ref.at[pl.ds(g * _CE, _CE)])
            return carry

        jax.lax.fori_loop(0, k_iters, step, 0)

    f = pl.kernel(
        body,
        out_type=[jax.ShapeDtypeStruct((e, d), table.dtype),
                  jax.ShapeDtypeStruct((e, d), table.dtype)],
        mesh=mesh,
        scratch_types=[
            pltpu.VMEM((_CH, _IDXW), jnp.int32),
            pltpu.VMEM((_CH, _IDXW), jnp.int32),
            pltpu.VMEM((_CE, d), table.dtype),
            pltpu.VMEM((_CE, d), table.dtype),
            pltpu.SemaphoreType.DMA,
            pltpu.SemaphoreType.DMA,
        ],
    )
    return f(table, dst3d, src3d)


def _sc_scatter(msg, dst3d, n):
    e, d = msg.shape
    n_chunks = e // _CE
    k_iters = -(-n_chunks // _NW)
    rpt = (n // 16) // 8 * 8
    tail = n - 16 * rpt
    zch = 208
    assert rpt % zch == 0 and tail <= zch
    mesh = plsc.VectorSubcoreMesh(core_axis_name="c", subcore_axis_name="s")

    def body(msg_ref, dst_ref, z_ref, part_ref, idx_v, rows_v, accum):
        cc = jax.lax.axis_index("c")
        ss = jax.lax.axis_index("s")
        wid = ss * 2 + cc

        def striped(fn):
            for i in range(rpt // zch):
                fn(ss * rpt + i * zch, zch)
            if tail:
                @pl.when(ss == 15)
                def _():
                    fn(16 * rpt, tail)

        pltpu.sync_copy(z_ref, rows_v.at[pl.ds(0, zch)])
        striped(lambda at, ln: pltpu.sync_copy(
            rows_v.at[pl.ds(0, ln)], accum.at[pl.ds(at, ln)]))
        plsc.subcore_barrier()

        def step(k, carry):
            g = wid + _NW * k

            @pl.when(g < n_chunks)
            def _():
                pltpu.sync_copy(dst_ref.at[g], idx_v)
                pltpu.sync_copy(msg_ref.at[pl.ds(g * _CE, _CE)], rows_v)
                for j in range(_CH):
                    pltpu.sync_copy(rows_v.at[pl.ds(j * _IDXW, _IDXW)],
                                    accum.at[idx_v.at[j]], add=True)
            return carry

        jax.lax.fori_loop(0, k_iters, step, 0)
        plsc.subcore_barrier()

        def wb(at, ln):
            pltpu.sync_copy(accum.at[pl.ds(at, ln)], rows_v.at[pl.ds(0, ln)])
            pltpu.sync_copy(rows_v.at[pl.ds(0, ln)],
                            part_ref.at[pl.ds(cc * n + at, ln)])
        striped(wb)

    f = pl.kernel(
        body,
        out_type=[jax.ShapeDtypeStruct((2 * n, d), jnp.float32)],
        mesh=mesh,
        scratch_types=[
            pltpu.VMEM((_CH, _IDXW), jnp.int32),
            pltpu.VMEM((_CE, d), jnp.float32),
            pltpu.VMEM_SHARED((n, d), jnp.float32),
        ],
    )
    (out,) = f(msg, dst3d, jnp.zeros((zch, d), jnp.float32))
    return out.reshape(2, n, d)


def kernel(x, edge_index, bn_gamma, bn_beta, eW1, eb1, eW2, eb2, eW3, eb3,
           dW1, db1, dW2, db2, dW3, db3):
    n, d = x.shape
    hid = eW3.shape[1]
    src = edge_index[0].reshape(-1, _CH, _IDXW)
    dst = edge_index[1].reshape(-1, _CH, _IDXW)

    eW1p = jnp.concatenate([eW1[:d] - eW1[d:], eW1[d:]], axis=0)
    dW1p = jnp.concatenate([dW1[:hid] - dW1[hid:], dW1[hid:]], axis=0)

    z = _batchnorm(x, bn_gamma, bn_beta)

    xi, xj = _sc_gather_pair(z, dst, src)
    m1 = _edge_mlp(xi, xj, eW1p, eb1, eW2, eb2, eW3, eb3, last_relu=True)
    parts1 = _sc_scatter(m1, dst, n)
    y = _combine(parts1, parts1, keep=hid)

    yi, yj = _sc_gather_pair(y, dst, src)
    m2 = _edge_mlp(yi, yj, dW1p, db1, dW2, db2, dW3, db3, last_relu=False)
    parts2 = _sc_scatter(m2, dst, n)
    return _combine(parts2, parts1, keep=d)


# speedup vs baseline: 1.7188x; 1.2117x over previous
import functools

import jax
import jax.numpy as jnp
from jax.experimental import pallas as pl
from jax.experimental.pallas import tpu as pltpu
from jax.experimental.pallas import tpu_sc as plsc

_EPS = 1e-5
_NW = 32
_IDXW = 128
_CH = 2
_CE = _CH * _IDXW
_LANES = 128


def _bn_body(x_ref, g_ref, b_ref, z_ref):
    x = x_ref[...]
    n = x.shape[0]
    mu = jnp.sum(x, axis=0, keepdims=True) / n
    xc = x - mu
    var = jnp.sum(xc * xc, axis=0, keepdims=True) / n
    z_ref[...] = g_ref[...] * xc * jax.lax.rsqrt(var + _EPS) + b_ref[...]


def _batchnorm(x, gamma, beta):
    n, d = x.shape
    return pl.pallas_call(
        _bn_body,
        out_shape=jax.ShapeDtypeStruct((n, d), jnp.float32),
    )(x, gamma.reshape(1, d), beta.reshape(1, d))


def _mlp_body(xi_ref, xj_ref, w1_ref, b1_ref, w2_ref, b2_ref, w3_ref, b3_ref,
              o_ref, *, last_relu):
    din = w1_ref.shape[0] // 2
    dout = w3_ref.shape[1]
    be = xi_ref.shape[0]

    def mm(a, w):
        if a.dtype != jnp.bfloat16:
            a = a.astype(jnp.bfloat16)
        return jnp.dot(a, w, preferred_element_type=jnp.float32)

    h = jnp.concatenate([xi_ref[...][:, :din], xj_ref[...][:, :din]], axis=1)
    h = jnp.maximum(mm(h, w1_ref[...]) + b1_ref[...], 0.0)
    h = jnp.maximum(mm(h, w2_ref[...]) + b2_ref[...], 0.0)
    o = mm(h, w3_ref[...]) + b3_ref[...]
    if last_relu:
        o = jnp.maximum(o, 0.0)
    if dout < _LANES:
        o = jnp.concatenate(
            [o, jnp.full((be, 8), 1.0, jnp.float32),
             jnp.zeros((be, _LANES - dout - 8), jnp.float32)], axis=1)
    o_ref[...] = o


def _edge_mlp(xi, xj, w1, b1, w2, b2, w3, b3, last_relu, be=2048):
    e, din = xi.shape
    big = w1.shape[1]
    dout = w3.shape[1]
    w1 = w1.astype(jnp.bfloat16)
    w2 = w2.astype(jnp.bfloat16)
    w3 = w3.astype(jnp.bfloat16)
    b1 = b1.reshape(1, big)
    b2 = b2.reshape(1, big)
    b3 = b3.reshape(1, dout)
    grid = e // be
    wspec = lambda a: pl.BlockSpec(a.shape, lambda i: (0, 0))
    return pl.pallas_call(
        functools.partial(_mlp_body, last_relu=last_relu),
        grid=(grid,),
        in_specs=[
            pl.BlockSpec((be, din), lambda i: (i, 0)),
            pl.BlockSpec((be, din), lambda i: (i, 0)),
            wspec(w1), wspec(b1), wspec(w2), wspec(b2), wspec(w3), wspec(b3),
        ],
        out_specs=pl.BlockSpec((be, _LANES), lambda i: (i, 0)),
        out_shape=jax.ShapeDtypeStruct((e, _LANES), jnp.float32),
        compiler_params=pltpu.CompilerParams(
            dimension_semantics=("arbitrary",)),
    )(xi, xj, w1, b1, w2, b2, w3, b3)


def _combine_body(pa_ref, pb_ref, ca_ref, cb_ref, y_ref, *, keep):
    pa = pa_ref[...]
    pb = pb_ref[...]
    psum = pa[0] + pa[1] + pb[0] + pb[1]
    c = ca_ref[...][0] + ca_ref[...][1] + cb_ref[...][0] + cb_ref[...][1]
    cnt = c[:, 64:65]
    y = psum / jnp.maximum(cnt, 1.0)
    if keep < _LANES:
        y = jnp.concatenate(
            [y[:, :keep], jnp.zeros((y.shape[0], _LANES - keep), y.dtype)],
            axis=1)
    y_ref[...] = y.astype(y_ref.dtype)


def _combine(parts_a, parts_b, cnt_a, cnt_b, keep, out_dtype=jnp.float32):
    _, n, d = parts_a.shape
    return pl.pallas_call(
        functools.partial(_combine_body, keep=keep),
        out_shape=jax.ShapeDtypeStruct((n, d), out_dtype),
    )(parts_a, parts_b, cnt_a, cnt_b)


def _sc_gather_pair(table, dst3d, src3d):
    n, d = table.shape
    n_chunks = dst3d.shape[0]
    e = n_chunks * _CE
    k_iters = -(-n_chunks // _NW)
    mesh = plsc.VectorSubcoreMesh(core_axis_name="c", subcore_axis_name="s")

    def body(table_ref, dst_ref, src_ref, xi_ref, xj_ref,
             idx_i, idx_j, rows_i, rows_j, sem_i, sem_j):
        cc = jax.lax.axis_index("c")
        ss = jax.lax.axis_index("s")
        wid = ss * 2 + cc

        def step(k, carry):
            g = wid + _NW * k

            @pl.when(g < n_chunks)
            def _():
                pltpu.sync_copy(dst_ref.at[g], idx_i)
                pltpu.sync_copy(src_ref.at[g], idx_j)
                cps = []
                for j in range(_CH):
                    cps.append(pltpu.async_copy(
                        table_ref.at[idx_i.at[j]],
                        rows_i.at[pl.ds(j * _IDXW, _IDXW)], sem_i))
                    cps.append(pltpu.async_copy(
                        table_ref.at[idx_j.at[j]],
                        rows_j.at[pl.ds(j * _IDXW, _IDXW)], sem_j))
                for cp in cps:
                    cp.wait()
                pltpu.sync_copy(rows_i, xi_ref.at[pl.ds(g * _CE, _CE)])
                pltpu.sync_copy(rows_j, xj_ref.at[pl.ds(g * _CE, _CE)])
            return carry

        jax.lax.fori_loop(0, k_iters, step, 0)

    f = pl.kernel(
        body,
        out_type=[jax.ShapeDtypeStruct((e, d), table.dtype),
                  jax.ShapeDtypeStruct((e, d), table.dtype)],
        mesh=mesh,
        scratch_types=[
            pltpu.VMEM((_CH, _IDXW), jnp.int32),
            pltpu.VMEM((_CH, _IDXW), jnp.int32),
            pltpu.VMEM((_CE, d), table.dtype),
            pltpu.VMEM((_CE, d), table.dtype),
            pltpu.SemaphoreType.DMA,
            pltpu.SemaphoreType.DMA,
        ],
    )
    return f(table, dst3d, src3d)


def _sc_scatter(msg, dst3d, n):
    e, d = msg.shape
    n_chunks = e // _CE
    k_iters = -(-n_chunks // _NW)
    rpt = (n // 16) // 8 * 8
    tail = n - 16 * rpt
    zch = 208
    assert rpt % zch == 0 and tail <= zch
    mesh = plsc.VectorSubcoreMesh(core_axis_name="c", subcore_axis_name="s")

    def body(msg_ref, dst_ref, z_ref, part_ref, idx_v, rows_v, accum):
        cc = jax.lax.axis_index("c")
        ss = jax.lax.axis_index("s")
        wid = ss * 2 + cc

        def striped(fn):
            for i in range(rpt // zch):
                fn(ss * rpt + i * zch, zch)
            if tail:
                @pl.when(ss == 15)
                def _():
                    fn(16 * rpt, tail)

        pltpu.sync_copy(z_ref, rows_v.at[pl.ds(0, zch)])
        striped(lambda at, ln: pltpu.sync_copy(
            rows_v.at[pl.ds(0, ln)], accum.at[pl.ds(at, ln)]))
        plsc.subcore_barrier()

        def step(k, carry):
            g = wid + _NW * k

            @pl.when(g < n_chunks)
            def _():
                pltpu.sync_copy(dst_ref.at[g], idx_v)
                pltpu.sync_copy(msg_ref.at[pl.ds(g * _CE, _CE)], rows_v)
                for j in range(_CH):
                    pltpu.sync_copy(rows_v.at[pl.ds(j * _IDXW, _IDXW)],
                                    accum.at[idx_v.at[j]], add=True)
            return carry

        jax.lax.fori_loop(0, k_iters, step, 0)
        plsc.subcore_barrier()

        def wb(at, ln):
            pltpu.sync_copy(accum.at[pl.ds(at, ln)], rows_v.at[pl.ds(0, ln)])
            pltpu.sync_copy(rows_v.at[pl.ds(0, ln)],
                            part_ref.at[pl.ds(cc * n + at, ln)])
        striped(wb)

    f = pl.kernel(
        body,
        out_type=[jax.ShapeDtypeStruct((2 * n, d), jnp.float32)],
        mesh=mesh,
        scratch_types=[
            pltpu.VMEM((_CH, _IDXW), jnp.int32),
            pltpu.VMEM((_CE, d), jnp.float32),
            pltpu.VMEM_SHARED((n, d), jnp.float32),
        ],
    )
    (out,) = f(msg, dst3d, jnp.zeros((zch, d), jnp.float32))
    return out.reshape(2, n, d)


def kernel(x, edge_index, bn_gamma, bn_beta, eW1, eb1, eW2, eb2, eW3, eb3,
           dW1, db1, dW2, db2, dW3, db3):
    n, d = x.shape
    hid = eW3.shape[1]
    src = edge_index[0].reshape(-1, _CH, _IDXW)
    dst = edge_index[1].reshape(-1, _CH, _IDXW)

    eW1p = jnp.concatenate([eW1[:d] - eW1[d:], eW1[d:]], axis=0)
    dW1p = jnp.concatenate([dW1[:hid] - dW1[hid:], dW1[hid:]], axis=0)

    z = _batchnorm(x, bn_gamma, bn_beta)

    nh = dst.shape[0] // 2
    halves = [(dst[:nh], src[:nh]), (dst[nh:], src[nh:])]

    def conv(table, w1, b1, w2, b2, w3, b3, last_relu):
        parts = []
        for dh, sh in halves:
            xi, xj = _sc_gather_pair(table, dh, sh)
            m = _edge_mlp(xi, xj, w1, b1, w2, b2, w3, b3,
                          last_relu=last_relu, be=2000)
            parts.append(_sc_scatter(m, dh, n))
        return parts

    p1a, p1b = conv(z, eW1p, eb1, eW2, eb2, eW3, eb3, True)
    y = _combine(p1a, p1b, p1a, p1b, keep=hid)
    p2a, p2b = conv(y, dW1p, db1, dW2, db2, dW3, db3, False)
    return _combine(p2a, p2b, p1a, p1b, keep=d)
